# Initial kernel scaffold; baseline (speedup 1.0000x reference)
import jax
import jax.numpy as jnp
from jax.experimental import pallas as pl


def kernel(x, edge_index, W1, b1, W2, b2, Wp, bp):
    raise NotImplementedError("write your pallas kernel here")


# speedup vs baseline: 14.8259x; 14.8259x over previous
import functools

import jax
import jax.numpy as jnp
from jax import lax
from jax.experimental import pallas as pl
from jax.experimental.pallas import tpu as pltpu
from jax.experimental.pallas import tpu_sc as plsc

N = 10000
E = 320000
D = 128
H = 128

NC = 2
NS = 16
L = 16
NW = NC * NS
EPT = E // NW
CHUNK = 80
NCHUNK = EPT // CHUNK
STRIPE = 624
TAIL = N - STRIPE * NS
TAIL_OFF = STRIPE * NS

BN = 1000
GRID = N // BN

_mesh = plsc.VectorSubcoreMesh(core_axis_name="c", subcore_axis_name="s")
_sc_params = pltpu.CompilerParams(needs_layout_passes=False)


@functools.partial(
    pl.kernel,
    out_type=jax.ShapeDtypeStruct((NW * N,), jnp.float32),
    mesh=_mesh,
    compiler_params=_sc_params,
    scratch_types=[
        pltpu.VMEM((N,), jnp.float32),
        pltpu.VMEM((EPT,), jnp.int32),
    ],
)
def _deg_kernel(dst_hbm, deg_out, deg_v, dstbuf):
    c = lax.axis_index("c")
    s = lax.axis_index("s")
    wid = s * NC + c
    pltpu.sync_copy(dst_hbm.at[pl.ds(wid * EPT, EPT)], dstbuf)
    zeros = jnp.zeros((L,), jnp.float32)

    def zbody(i, carry):
        deg_v[pl.ds(i * L, L)] = zeros
        return carry

    lax.fori_loop(0, N // L, zbody, 0)
    ones = jnp.ones((L,), jnp.float32)

    def body(i, carry):
        idx = dstbuf[pl.ds(i * L, L)]
        plsc.addupdate_scatter(deg_v, [idx], ones)
        return carry

    lax.fori_loop(0, EPT // L, body, 0)
    pltpu.sync_copy(deg_v, deg_out.at[pl.ds(wid * N, N)])


def _prep_body(degt_ref, x_ref, w1_ref, norm_ref, yhat_ref):
    deg = jnp.sum(degt_ref[...], axis=1) + 1.0
    nrm = lax.rsqrt(deg)
    norm_ref[...] = nrm[:, None]
    y = jnp.dot(x_ref[...], w1_ref[...], preferred_element_type=jnp.float32)
    yhat_ref[...] = y * nrm[:, None]


_prep_call = pl.pallas_call(
    _prep_body,
    grid=(GRID,),
    in_specs=[
        pl.BlockSpec((BN, NW), lambda i: (i, 0)),
        pl.BlockSpec((BN, D), lambda i: (i, 0)),
        pl.BlockSpec((D, H), lambda i: (0, 0)),
    ],
    out_specs=[
        pl.BlockSpec((BN, 1), lambda i: (i, 0)),
        pl.BlockSpec((BN, H), lambda i: (i, 0)),
    ],
    out_shape=[
        jax.ShapeDtypeStruct((N, 1), jnp.float32),
        jax.ShapeDtypeStruct((N, H), jnp.float32),
    ],
)


@functools.partial(
    pl.kernel,
    out_type=[
        jax.ShapeDtypeStruct((NC, N, H), jnp.float32),
        jax.ShapeDtypeStruct((NW * N,), jnp.float32),
    ],
    mesh=_mesh,
    compiler_params=_sc_params,
    scratch_types=[
        pltpu.VMEM_SHARED((N, H), jnp.float32),
        pltpu.VMEM((N,), jnp.float32),
        pltpu.VMEM((N,), jnp.float32),
        pltpu.VMEM((CHUNK,), jnp.int32),
        pltpu.VMEM((CHUNK,), jnp.int32),
        pltpu.VMEM((CHUNK, H), jnp.float32),
        pltpu.SemaphoreType.DMA,
    ],
)
def _main_kernel(yhat_hbm, norm_hbm, src_hbm, dst_hbm, S_out, s_out,
                 acc_sh, norm_v, s_v, srcbuf, dstbuf, rows, sem):
    c = lax.axis_index("c")
    s = lax.axis_index("s")
    wid = s * NC + c
    off = pl.multiple_of(s * STRIPE, 8)
    pltpu.sync_copy(yhat_hbm.at[pl.ds(off, STRIPE)],
                    acc_sh.at[pl.ds(off, STRIPE)])

    @pl.when(s == 0)
    def _():
        pltpu.sync_copy(yhat_hbm.at[pl.ds(TAIL_OFF, TAIL)],
                        acc_sh.at[pl.ds(TAIL_OFF, TAIL)])
    pltpu.sync_copy(norm_hbm, norm_v)
    zeros = jnp.zeros((L,), jnp.float32)

    def zbody(i, carry):
        s_v[pl.ds(i * L, L)] = zeros
        return carry

    lax.fori_loop(0, N // L, zbody, 0)
    plsc.subcore_barrier()

    ebase = wid * EPT

    def chunk_body(i, carry):
        base = ebase + i * CHUNK
        pltpu.sync_copy(src_hbm.at[pl.ds(base, CHUNK)], srcbuf)
        pltpu.sync_copy(dst_hbm.at[pl.ds(base, CHUNK)], dstbuf)
        pltpu.async_copy(yhat_hbm.at[srcbuf], rows, sem).wait()
        pltpu.sync_copy(rows, acc_sh.at[dstbuf], add=True)
        for j in range(CHUNK // L):
            d16 = dstbuf[pl.ds(j * L, L)]
            s16 = srcbuf[pl.ds(j * L, L)]
            val = plsc.load_gather(norm_v, [d16])
            plsc.addupdate_scatter(s_v, [s16], val)
        return carry

    lax.fori_loop(0, NCHUNK, chunk_body, 0)
    plsc.subcore_barrier()
    pltpu.sync_copy(acc_sh.at[pl.ds(off, STRIPE)],
                    S_out.at[c, pl.ds(off, STRIPE)])

    @pl.when(s == 0)
    def _():
        pltpu.sync_copy(acc_sh.at[pl.ds(TAIL_OFF, TAIL)],
                        S_out.at[c, pl.ds(TAIL_OFF, TAIL)])

    pltpu.sync_copy(s_v, s_out.at[pl.ds(wid * N, N)])


def _final_body(S_ref, yhat_ref, norm_ref, sp_ref, b1_ref, w2_ref, b2_ref,
                wp_ref, bp_ref, out_ref, acc):
    i = pl.program_id(0)

    @pl.when(i == 0)
    def _():
        acc[...] = jnp.zeros_like(acc)

    nrm = norm_ref[...]
    ssum = S_ref[0] + S_ref[1] - yhat_ref[...]
    agg = ssum * nrm + b1_ref[...]
    h = jnp.maximum(agg, 0.0)
    stot = jnp.sum(sp_ref[...], axis=1)
    cvec = nrm[:, 0] * (nrm[:, 0] + stot)
    acc[...] += jnp.dot(cvec[None, :], h, preferred_element_type=jnp.float32)

    @pl.when(i == pl.num_programs(0) - 1)
    def _():
        graph = jnp.dot(acc[...] / N, w2_ref[...],
                        preferred_element_type=jnp.float32) + b2_ref[...]
        proj = jnp.dot(graph, wp_ref[...],
                       preferred_element_type=jnp.float32) + bp_ref[...]
        nn = jnp.sqrt(jnp.sum(proj * proj))
        out_ref[...] = proj / jnp.maximum(nn, 1e-12)


_final_call = pl.pallas_call(
    _final_body,
    grid=(GRID,),
    in_specs=[
        pl.BlockSpec((NC, BN, H), lambda i: (0, i, 0)),
        pl.BlockSpec((BN, H), lambda i: (i, 0)),
        pl.BlockSpec((BN, 1), lambda i: (i, 0)),
        pl.BlockSpec((BN, NW), lambda i: (i, 0)),
        pl.BlockSpec((1, H), lambda i: (0, 0)),
        pl.BlockSpec((H, H), lambda i: (0, 0)),
        pl.BlockSpec((1, H), lambda i: (0, 0)),
        pl.BlockSpec((H, H), lambda i: (0, 0)),
        pl.BlockSpec((1, H), lambda i: (0, 0)),
    ],
    out_specs=pl.BlockSpec((1, H), lambda i: (0, 0)),
    out_shape=jax.ShapeDtypeStruct((1, H), jnp.float32),
    scratch_shapes=[pltpu.VMEM((1, H), jnp.float32)],
)


def kernel(x, edge_index, W1, b1, W2, b2, Wp, bp):
    src = edge_index[0]
    dst = edge_index[1]
    deg_lin = _deg_kernel(dst)
    degT = deg_lin.reshape(NW, N).T
    norm_col, yhat = _prep_call(degT, x, W1)
    S_parts, s_lin = _main_kernel(yhat, norm_col.reshape(N), src, dst)
    sT = s_lin.reshape(NW, N).T
    return _final_call(S_parts, yhat, norm_col, sT,
                       b1.reshape(1, H), W2, b2.reshape(1, H),
                       Wp, bp.reshape(1, H))


# speedup vs baseline: 26.6751x; 1.7992x over previous
import functools

import jax
import jax.numpy as jnp
from jax import lax
from jax.experimental import pallas as pl
from jax.experimental.pallas import tpu as pltpu
from jax.experimental.pallas import tpu_sc as plsc

N = 10000
E = 320000
D = 128
H = 128

NC = 2
NS = 16
L = 16
NW = NC * NS
EPT = E // NW
CHUNK = 80
NCHUNK = EPT // CHUNK
STRIPE = 624
TAIL = N - STRIPE * NS
TAIL_OFF = STRIPE * NS

ECHUNK = 96
NCH = 104
E_MAIN = NW * NCH * ECHUNK
TAIL_PT = (E - E_MAIN) // NW

BN = 1000
GRID = N // BN

_mesh = plsc.VectorSubcoreMesh(core_axis_name="c", subcore_axis_name="s")
_sc_params = pltpu.CompilerParams(needs_layout_passes=False)


@functools.partial(
    pl.kernel,
    out_type=jax.ShapeDtypeStruct((NW * N,), jnp.float32),
    mesh=_mesh,
    compiler_params=_sc_params,
    scratch_types=[
        pltpu.VMEM((N,), jnp.float32),
        pltpu.VMEM((EPT,), jnp.int32),
    ],
)
def _deg_kernel(dst_hbm, deg_out, deg_v, dstbuf):
    c = lax.axis_index("c")
    s = lax.axis_index("s")
    wid = s * NC + c
    pltpu.sync_copy(dst_hbm.at[pl.ds(wid * EPT, EPT)], dstbuf)
    zeros = jnp.zeros((L,), jnp.float32)

    def zbody(i, carry):
        deg_v[pl.ds(i * L, L)] = zeros
        return carry

    lax.fori_loop(0, N // L, zbody, 0)
    ones = jnp.ones((L,), jnp.float32)

    def body(i, carry):
        idx = dstbuf[pl.ds(i * L, L)]
        plsc.addupdate_scatter(deg_v, [idx], ones)
        return carry

    lax.fori_loop(0, EPT // L, body, 0)
    pltpu.sync_copy(deg_v, deg_out.at[pl.ds(wid * N, N)])


def _prep_body(degt_ref, x_ref, w1_ref, norm_ref, yhat_ref):
    deg = jnp.sum(degt_ref[...], axis=1) + 1.0
    nrm = lax.rsqrt(deg)
    norm_ref[...] = nrm[:, None]
    y = jnp.dot(x_ref[...], w1_ref[...], preferred_element_type=jnp.float32)
    yhat_ref[...] = y * nrm[:, None]


_prep_call = pl.pallas_call(
    _prep_body,
    grid=(GRID,),
    in_specs=[
        pl.BlockSpec((BN, NW), lambda i: (i, 0)),
        pl.BlockSpec((BN, D), lambda i: (i, 0)),
        pl.BlockSpec((D, H), lambda i: (0, 0)),
    ],
    out_specs=[
        pl.BlockSpec((BN, 1), lambda i: (i, 0)),
        pl.BlockSpec((BN, H), lambda i: (i, 0)),
    ],
    out_shape=[
        jax.ShapeDtypeStruct((N, 1), jnp.float32),
        jax.ShapeDtypeStruct((N, H), jnp.float32),
    ],
)


@functools.partial(
    pl.kernel,
    out_type=[
        jax.ShapeDtypeStruct((NC, N, H), jnp.float32),
        jax.ShapeDtypeStruct((NW * N,), jnp.float32),
    ],
    mesh=_mesh,
    compiler_params=_sc_params,
    scratch_types=[
        pltpu.VMEM_SHARED((N, H), jnp.float32),
        pltpu.VMEM((N,), jnp.float32),
        pltpu.VMEM((N,), jnp.float32),
        [pltpu.VMEM((ECHUNK,), jnp.int32)] * 2,
        [pltpu.VMEM((ECHUNK,), jnp.int32)] * 2,
        [pltpu.VMEM((ECHUNK, H), jnp.float32)] * 2,
        pltpu.VMEM((TAIL_PT,), jnp.int32),
        pltpu.VMEM((TAIL_PT,), jnp.int32),
        [pltpu.SemaphoreType.DMA] * 2,
        [pltpu.SemaphoreType.DMA] * 2,
        [pltpu.SemaphoreType.DMA] * 2,
        [pltpu.SemaphoreType.DMA] * 2,
    ],
)
def _main_kernel(yhat_hbm, norm_hbm, src_hbm, dst_hbm, S_out, s_out,
                 acc_sh, norm_v, s_v, srcb, dstb, rows, tsrc, tdst,
                 sem_si, sem_di, sem_g, sem_s):
    c = lax.axis_index("c")
    s = lax.axis_index("s")
    wid = s * NC + c
    ebase = wid * (NCH * ECHUNK)

    def idx_start(j, b):
        base = ebase + j * ECHUNK
        pltpu.async_copy(src_hbm.at[pl.ds(base, ECHUNK)], srcb[b], sem_si[b])
        pltpu.async_copy(dst_hbm.at[pl.ds(base, ECHUNK)], dstb[b], sem_di[b])

    def idx_wait(j, b):
        base = ebase + j * ECHUNK
        pltpu.make_async_copy(src_hbm.at[pl.ds(base, ECHUNK)], srcb[b],
                              sem_si[b]).wait()
        pltpu.make_async_copy(dst_hbm.at[pl.ds(base, ECHUNK)], dstb[b],
                              sem_di[b]).wait()

    def gather_start(b):
        pltpu.async_copy(yhat_hbm.at[srcb[b]], rows[b], sem_g[b])

    def gather_wait(b):
        pltpu.make_async_copy(yhat_hbm.at[srcb[b]], rows[b], sem_g[b]).wait()

    def scatter_start(b):
        pltpu.async_copy(rows[b], acc_sh.at[dstb[b]], sem_s[b], add=True)

    def scatter_wait(b):
        pltpu.make_async_copy(rows[b], acc_sh.at[dstb[b]], sem_s[b]).wait()

    def s_pass(b):
        for g in range(ECHUNK // L):
            d16 = dstb[b][pl.ds(g * L, L)]
            s16 = srcb[b][pl.ds(g * L, L)]
            val = plsc.load_gather(norm_v, [d16])
            plsc.addupdate_scatter(s_v, [s16], val)

    idx_start(0, 0)
    idx_start(1, 1)
    off = pl.multiple_of(s * STRIPE, 8)
    pltpu.sync_copy(yhat_hbm.at[pl.ds(off, STRIPE)],
                    acc_sh.at[pl.ds(off, STRIPE)])

    @pl.when(s == 0)
    def _():
        pltpu.sync_copy(yhat_hbm.at[pl.ds(TAIL_OFF, TAIL)],
                        acc_sh.at[pl.ds(TAIL_OFF, TAIL)])
    pltpu.sync_copy(norm_hbm, norm_v)
    zeros = jnp.zeros((L,), jnp.float32)

    def zbody(i, carry):
        s_v[pl.ds(i * L, L)] = zeros
        return carry

    lax.fori_loop(0, N // L, zbody, 0)
    plsc.subcore_barrier()

    idx_wait(0, 0)
    gather_start(0)

    def pair_body(p, carry):
        for b in range(2):
            j = 2 * p + b
            nb = 1 - b
            idx_wait(j + 1, nb)
            gather_start(nb)
            gather_wait(b)
            scatter_start(b)
            s_pass(b)
            scatter_wait(b)

            @pl.when(j + 2 < NCH)
            def _():
                idx_start(j + 2, b)
        return carry

    lax.fori_loop(0, NCH // 2 - 1, pair_body, 0)

    for b in range(2):
        j = NCH - 2 + b
        if b == 0:
            idx_wait(j + 1, 1)
            gather_start(1)
        gather_wait(b)
        scatter_start(b)
        s_pass(b)
        scatter_wait(b)

    tbase = E_MAIN + wid * TAIL_PT
    pltpu.sync_copy(src_hbm.at[pl.ds(tbase, TAIL_PT)], tsrc)
    pltpu.sync_copy(dst_hbm.at[pl.ds(tbase, TAIL_PT)], tdst)
    pltpu.async_copy(yhat_hbm.at[tsrc], rows[0].at[pl.ds(0, TAIL_PT)],
                     sem_g[0]).wait()
    pltpu.sync_copy(rows[0].at[pl.ds(0, TAIL_PT)], acc_sh.at[tdst], add=True)
    td16 = tdst[pl.ds(0, L)]
    ts16 = tsrc[pl.ds(0, L)]
    tval = plsc.load_gather(norm_v, [td16])
    plsc.addupdate_scatter(s_v, [ts16], tval)

    plsc.subcore_barrier()
    pltpu.sync_copy(acc_sh.at[pl.ds(off, STRIPE)],
                    S_out.at[c, pl.ds(off, STRIPE)])

    @pl.when(s == 0)
    def _():
        pltpu.sync_copy(acc_sh.at[pl.ds(TAIL_OFF, TAIL)],
                        S_out.at[c, pl.ds(TAIL_OFF, TAIL)])

    pltpu.sync_copy(s_v, s_out.at[pl.ds(wid * N, N)])


def _final_body(S_ref, yhat_ref, norm_ref, sp_ref, b1_ref, w2_ref, b2_ref,
                wp_ref, bp_ref, out_ref, acc):
    i = pl.program_id(0)

    @pl.when(i == 0)
    def _():
        acc[...] = jnp.zeros_like(acc)

    nrm = norm_ref[...]
    ssum = S_ref[0] + S_ref[1] - yhat_ref[...]
    agg = ssum * nrm + b1_ref[...]
    h = jnp.maximum(agg, 0.0)
    stot = jnp.sum(sp_ref[...], axis=1)
    cvec = nrm[:, 0] * (nrm[:, 0] + stot)
    acc[...] += jnp.dot(cvec[None, :], h, preferred_element_type=jnp.float32)

    @pl.when(i == pl.num_programs(0) - 1)
    def _():
        graph = jnp.dot(acc[...] / N, w2_ref[...],
                        preferred_element_type=jnp.float32) + b2_ref[...]
        proj = jnp.dot(graph, wp_ref[...],
                       preferred_element_type=jnp.float32) + bp_ref[...]
        nn = jnp.sqrt(jnp.sum(proj * proj))
        out_ref[...] = proj / jnp.maximum(nn, 1e-12)


_final_call = pl.pallas_call(
    _final_body,
    grid=(GRID,),
    in_specs=[
        pl.BlockSpec((NC, BN, H), lambda i: (0, i, 0)),
        pl.BlockSpec((BN, H), lambda i: (i, 0)),
        pl.BlockSpec((BN, 1), lambda i: (i, 0)),
        pl.BlockSpec((BN, NW), lambda i: (i, 0)),
        pl.BlockSpec((1, H), lambda i: (0, 0)),
        pl.BlockSpec((H, H), lambda i: (0, 0)),
        pl.BlockSpec((1, H), lambda i: (0, 0)),
        pl.BlockSpec((H, H), lambda i: (0, 0)),
        pl.BlockSpec((1, H), lambda i: (0, 0)),
    ],
    out_specs=pl.BlockSpec((1, H), lambda i: (0, 0)),
    out_shape=jax.ShapeDtypeStruct((1, H), jnp.float32),
    scratch_shapes=[pltpu.VMEM((1, H), jnp.float32)],
)


def kernel(x, edge_index, W1, b1, W2, b2, Wp, bp):
    src = edge_index[0]
    dst = edge_index[1]
    deg_lin = _deg_kernel(dst)
    degT = deg_lin.reshape(NW, N).T
    norm_col, yhat = _prep_call(degT, x, W1)
    S_parts, s_lin = _main_kernel(yhat, norm_col.reshape(N), src, dst)
    sT = s_lin.reshape(NW, N).T
    return _final_call(S_parts, yhat, norm_col, sT,
                       b1.reshape(1, H), W2, b2.reshape(1, H),
                       Wp, bp.reshape(1, H))


# speedup vs baseline: 27.9297x; 1.0470x over previous
import functools

import jax
import jax.numpy as jnp
from jax import lax
from jax.experimental import pallas as pl
from jax.experimental.pallas import tpu as pltpu
from jax.experimental.pallas import tpu_sc as plsc

N = 10000
E = 320000
D = 128
H = 128

NC = 2
NS = 16
L = 16
NW = NC * NS
EPT = E // NW
CHUNK = 80
NCHUNK = EPT // CHUNK
STRIPE = 624
TAIL = N - STRIPE * NS
TAIL_OFF = STRIPE * NS

ECHUNK = 48
NCH = 208
NB = 4
E_MAIN = NW * NCH * ECHUNK
TAIL_PT = (E - E_MAIN) // NW

BN = 1000
GRID = N // BN

_mesh = plsc.VectorSubcoreMesh(core_axis_name="c", subcore_axis_name="s")
_sc_params = pltpu.CompilerParams(needs_layout_passes=False)


@functools.partial(
    pl.kernel,
    out_type=jax.ShapeDtypeStruct((NW * N,), jnp.float32),
    mesh=_mesh,
    compiler_params=_sc_params,
    scratch_types=[
        pltpu.VMEM((N,), jnp.float32),
        pltpu.VMEM((EPT,), jnp.int32),
    ],
)
def _deg_kernel(dst_hbm, deg_out, deg_v, dstbuf):
    c = lax.axis_index("c")
    s = lax.axis_index("s")
    wid = s * NC + c
    pltpu.sync_copy(dst_hbm.at[pl.ds(wid * EPT, EPT)], dstbuf)
    zeros = jnp.zeros((L,), jnp.float32)

    def zbody(i, carry):
        deg_v[pl.ds(i * L, L)] = zeros
        return carry

    lax.fori_loop(0, N // L, zbody, 0)
    ones = jnp.ones((L,), jnp.float32)

    def body(i, carry):
        idx = dstbuf[pl.ds(i * L, L)]
        plsc.addupdate_scatter(deg_v, [idx], ones)
        return carry

    lax.fori_loop(0, EPT // L, body, 0)
    pltpu.sync_copy(deg_v, deg_out.at[pl.ds(wid * N, N)])


def _prep_body(degt_ref, x_ref, w1_ref, norm_ref, yhat_ref):
    deg = jnp.sum(degt_ref[...], axis=1) + 1.0
    nrm = lax.rsqrt(deg)
    norm_ref[...] = nrm[:, None]
    y = jnp.dot(x_ref[...], w1_ref[...], preferred_element_type=jnp.float32)
    yhat_ref[...] = y * nrm[:, None]


_prep_call = pl.pallas_call(
    _prep_body,
    grid=(GRID,),
    in_specs=[
        pl.BlockSpec((BN, NW), lambda i: (i, 0)),
        pl.BlockSpec((BN, D), lambda i: (i, 0)),
        pl.BlockSpec((D, H), lambda i: (0, 0)),
    ],
    out_specs=[
        pl.BlockSpec((BN, 1), lambda i: (i, 0)),
        pl.BlockSpec((BN, H), lambda i: (i, 0)),
    ],
    out_shape=[
        jax.ShapeDtypeStruct((N, 1), jnp.float32),
        jax.ShapeDtypeStruct((N, H), jnp.float32),
    ],
)


@functools.partial(
    pl.kernel,
    out_type=[
        jax.ShapeDtypeStruct((NC, N, H), jnp.float32),
        jax.ShapeDtypeStruct((NW * N,), jnp.float32),
    ],
    mesh=_mesh,
    compiler_params=_sc_params,
    scratch_types=[
        pltpu.VMEM_SHARED((N, H), jnp.float32),
        pltpu.VMEM((N,), jnp.float32),
        pltpu.VMEM((N,), jnp.float32),
        [pltpu.VMEM((ECHUNK,), jnp.int32)] * NB,
        [pltpu.VMEM((ECHUNK,), jnp.int32)] * NB,
        [pltpu.VMEM((ECHUNK, H), jnp.float32)] * NB,
        pltpu.VMEM((TAIL_PT,), jnp.int32),
        pltpu.VMEM((TAIL_PT,), jnp.int32),
        [pltpu.SemaphoreType.DMA] * NB,
        [pltpu.SemaphoreType.DMA] * NB,
        [pltpu.SemaphoreType.DMA] * NB,
        [pltpu.SemaphoreType.DMA] * NB,
    ],
)
def _main_kernel(yhat_hbm, norm_hbm, src_hbm, dst_hbm, S_out, s_out,
                 acc_sh, norm_v, s_v, srcb, dstb, rows, tsrc, tdst,
                 sem_si, sem_di, sem_g, sem_s):
    c = lax.axis_index("c")
    s = lax.axis_index("s")
    wid = s * NC + c
    ebase = wid * (NCH * ECHUNK)

    def idx_start(j, b):
        base = ebase + j * ECHUNK
        pltpu.async_copy(src_hbm.at[pl.ds(base, ECHUNK)], srcb[b], sem_si[b])
        pltpu.async_copy(dst_hbm.at[pl.ds(base, ECHUNK)], dstb[b], sem_di[b])

    def idx_wait(j, b):
        base = ebase + j * ECHUNK
        pltpu.make_async_copy(src_hbm.at[pl.ds(base, ECHUNK)], srcb[b],
                              sem_si[b]).wait()
        pltpu.make_async_copy(dst_hbm.at[pl.ds(base, ECHUNK)], dstb[b],
                              sem_di[b]).wait()

    def gather_start(b):
        pltpu.async_copy(yhat_hbm.at[srcb[b]], rows[b], sem_g[b])

    def gather_wait(b):
        pltpu.make_async_copy(yhat_hbm.at[srcb[b]], rows[b], sem_g[b]).wait()

    def scatter_start(b):
        pltpu.async_copy(rows[b], acc_sh.at[dstb[b]], sem_s[b], add=True)

    def scatter_wait(b):
        pltpu.make_async_copy(rows[b], acc_sh.at[dstb[b]], sem_s[b]).wait()

    def s_pass(b):
        for g in range(ECHUNK // L):
            d16 = dstb[b][pl.ds(g * L, L)]
            s16 = srcb[b][pl.ds(g * L, L)]
            val = plsc.load_gather(norm_v, [d16])
            plsc.addupdate_scatter(s_v, [s16], val)

    idx_start(0, 0)
    idx_start(1, 1)
    off = pl.multiple_of(s * STRIPE, 8)
    pltpu.sync_copy(yhat_hbm.at[pl.ds(off, STRIPE)],
                    acc_sh.at[pl.ds(off, STRIPE)])

    @pl.when(s == 0)
    def _():
        pltpu.sync_copy(yhat_hbm.at[pl.ds(TAIL_OFF, TAIL)],
                        acc_sh.at[pl.ds(TAIL_OFF, TAIL)])
    pltpu.sync_copy(norm_hbm, norm_v)
    zeros = jnp.zeros((L,), jnp.float32)

    def zbody(i, carry):
        s_v[pl.ds(i * L, L)] = zeros
        return carry

    lax.fori_loop(0, N // L, zbody, 0)
    plsc.subcore_barrier()

    idx_wait(0, 0)
    gather_start(0)

    def quad_body(p, carry):
        for b in range(NB):
            j = NB * p + b
            b1 = (b + 1) % NB
            b2 = (b + 2) % NB

            @pl.when(j >= 2)
            def _():
                scatter_wait(b2)

            @pl.when(j + 2 < NCH)
            def _():
                idx_start(j + 2, b2)

            @pl.when(j + 1 < NCH)
            def _():
                idx_wait(j + 1, b1)
                gather_start(b1)

            gather_wait(b)
            scatter_start(b)
            s_pass(b)
        return carry

    lax.fori_loop(0, NCH // NB, quad_body, 0)

    scatter_wait((NCH - 2) % NB)
    scatter_wait((NCH - 1) % NB)

    tbase = E_MAIN + wid * TAIL_PT
    pltpu.sync_copy(src_hbm.at[pl.ds(tbase, TAIL_PT)], tsrc)
    pltpu.sync_copy(dst_hbm.at[pl.ds(tbase, TAIL_PT)], tdst)
    pltpu.async_copy(yhat_hbm.at[tsrc], rows[0].at[pl.ds(0, TAIL_PT)],
                     sem_g[0]).wait()
    pltpu.sync_copy(rows[0].at[pl.ds(0, TAIL_PT)], acc_sh.at[tdst], add=True)
    td16 = tdst[pl.ds(0, L)]
    ts16 = tsrc[pl.ds(0, L)]
    tval = plsc.load_gather(norm_v, [td16])
    plsc.addupdate_scatter(s_v, [ts16], tval)

    plsc.subcore_barrier()
    pltpu.sync_copy(acc_sh.at[pl.ds(off, STRIPE)],
                    S_out.at[c, pl.ds(off, STRIPE)])

    @pl.when(s == 0)
    def _():
        pltpu.sync_copy(acc_sh.at[pl.ds(TAIL_OFF, TAIL)],
                        S_out.at[c, pl.ds(TAIL_OFF, TAIL)])

    pltpu.sync_copy(s_v, s_out.at[pl.ds(wid * N, N)])


def _final_body(S_ref, yhat_ref, norm_ref, sp_ref, b1_ref, w2_ref, b2_ref,
                wp_ref, bp_ref, out_ref, acc):
    i = pl.program_id(0)

    @pl.when(i == 0)
    def _():
        acc[...] = jnp.zeros_like(acc)

    nrm = norm_ref[...]
    ssum = S_ref[0] + S_ref[1] - yhat_ref[...]
    agg = ssum * nrm + b1_ref[...]
    h = jnp.maximum(agg, 0.0)
    stot = jnp.sum(sp_ref[...], axis=1)
    cvec = nrm[:, 0] * (nrm[:, 0] + stot)
    acc[...] += jnp.dot(cvec[None, :], h, preferred_element_type=jnp.float32)

    @pl.when(i == pl.num_programs(0) - 1)
    def _():
        graph = jnp.dot(acc[...] / N, w2_ref[...],
                        preferred_element_type=jnp.float32) + b2_ref[...]
        proj = jnp.dot(graph, wp_ref[...],
                       preferred_element_type=jnp.float32) + bp_ref[...]
        nn = jnp.sqrt(jnp.sum(proj * proj))
        out_ref[...] = proj / jnp.maximum(nn, 1e-12)


_final_call = pl.pallas_call(
    _final_body,
    grid=(GRID,),
    in_specs=[
        pl.BlockSpec((NC, BN, H), lambda i: (0, i, 0)),
        pl.BlockSpec((BN, H), lambda i: (i, 0)),
        pl.BlockSpec((BN, 1), lambda i: (i, 0)),
        pl.BlockSpec((BN, NW), lambda i: (i, 0)),
        pl.BlockSpec((1, H), lambda i: (0, 0)),
        pl.BlockSpec((H, H), lambda i: (0, 0)),
        pl.BlockSpec((1, H), lambda i: (0, 0)),
        pl.BlockSpec((H, H), lambda i: (0, 0)),
        pl.BlockSpec((1, H), lambda i: (0, 0)),
    ],
    out_specs=pl.BlockSpec((1, H), lambda i: (0, 0)),
    out_shape=jax.ShapeDtypeStruct((1, H), jnp.float32),
    scratch_shapes=[pltpu.VMEM((1, H), jnp.float32)],
)


def kernel(x, edge_index, W1, b1, W2, b2, Wp, bp):
    src = edge_index[0]
    dst = edge_index[1]
    deg_lin = _deg_kernel(dst)
    degT = deg_lin.reshape(NW, N).T
    norm_col, yhat = _prep_call(degT, x, W1)
    S_parts, s_lin = _main_kernel(yhat, norm_col.reshape(N), src, dst)
    sT = s_lin.reshape(NW, N).T
    return _final_call(S_parts, yhat, norm_col, sT,
                       b1.reshape(1, H), W2, b2.reshape(1, H),
                       Wp, bp.reshape(1, H))


# speedup vs baseline: 28.2055x; 1.0099x over previous
import functools

import jax
import jax.numpy as jnp
from jax import lax
from jax.experimental import pallas as pl
from jax.experimental.pallas import tpu as pltpu
from jax.experimental.pallas import tpu_sc as plsc

N = 10000
E = 320000
D = 128
H = 128

NC = 2
NS = 16
L = 16
NW = NC * NS
EPT = E // NW
CHUNK = 80
NCHUNK = EPT // CHUNK
STRIPE = 624
TAIL = N - STRIPE * NS
TAIL_OFF = STRIPE * NS

CC = 128
RR = 80
NP = RR * CC

ECHUNK = 48
NCH = 208
NB = 4
E_MAIN = NW * NCH * ECHUNK
TAIL_PT = (E - E_MAIN) // NW

BN = 1000
GRID = N // BN

_mesh = plsc.VectorSubcoreMesh(core_axis_name="c", subcore_axis_name="s")
_sc_params = pltpu.CompilerParams(needs_layout_passes=False)


def _zero_2d(ref):
    zeros = jnp.zeros((L,), jnp.float32)

    def zbody(i, carry):
        ref[i >> 3, pl.ds((i & 7) * L, L)] = zeros
        return carry

    lax.fori_loop(0, RR * CC // L, zbody, 0)


def _rowadd_to_shared(vref, shref):
    for k in range(RR // L):
        rid = lax.iota(jnp.int32, L) + (k * L)
        pltpu.sync_copy(vref.at[pl.ds(k * L, L)], shref.at[rid], add=True)


def _readout_shared(shref, out_hbm, c, s):
    @pl.when(s < RR // 8)
    def _():
        pltpu.sync_copy(shref.at[pl.ds(s * 8, 8)],
                        out_hbm.at[pl.ds(c * RR + s * 8, 8)])


@functools.partial(
    pl.kernel,
    out_type=jax.ShapeDtypeStruct((NC * RR, CC), jnp.float32),
    mesh=_mesh,
    compiler_params=_sc_params,
    scratch_types=[
        pltpu.VMEM_SHARED((RR, CC), jnp.float32),
        pltpu.VMEM((RR, CC), jnp.float32),
        pltpu.VMEM((EPT,), jnp.int32),
    ],
)
def _deg_kernel(dst_hbm, deg_out, deg_sh, deg_v, dstbuf):
    c = lax.axis_index("c")
    s = lax.axis_index("s")
    wid = s * NC + c
    pltpu.sync_copy(dst_hbm.at[pl.ds(wid * EPT, EPT)], dstbuf)
    _zero_2d(deg_v)
    pltpu.sync_copy(deg_v.at[pl.ds(0, RR // NS)],
                    deg_sh.at[pl.ds(s * (RR // NS), RR // NS)])
    ones = jnp.ones((L,), jnp.float32)

    def body(i, carry):
        idx = dstbuf[pl.ds(i * L, L)]
        plsc.addupdate_scatter(deg_v, [idx >> 7, idx & 127], ones)
        return carry

    lax.fori_loop(0, EPT // L, body, 0)
    plsc.subcore_barrier()
    _rowadd_to_shared(deg_v, deg_sh)
    plsc.subcore_barrier()
    _readout_shared(deg_sh, deg_out, c, s)


def _prep_body(degt_ref, x_ref, w1_ref, norm_ref, yhat_ref):
    deg = jnp.sum(degt_ref[...], axis=1) + 1.0
    nrm = lax.rsqrt(deg)
    norm_ref[...] = nrm[:, None]
    y = jnp.dot(x_ref[...], w1_ref[...], preferred_element_type=jnp.float32)
    yhat_ref[...] = y * nrm[:, None]


_prep_call = pl.pallas_call(
    _prep_body,
    grid=(GRID,),
    in_specs=[
        pl.BlockSpec((BN, NC), lambda i: (i, 0)),
        pl.BlockSpec((BN, D), lambda i: (i, 0)),
        pl.BlockSpec((D, H), lambda i: (0, 0)),
    ],
    out_specs=[
        pl.BlockSpec((BN, 1), lambda i: (i, 0)),
        pl.BlockSpec((BN, H), lambda i: (i, 0)),
    ],
    out_shape=[
        jax.ShapeDtypeStruct((N, 1), jnp.float32),
        jax.ShapeDtypeStruct((N, H), jnp.float32),
    ],
)


@functools.partial(
    pl.kernel,
    out_type=[
        jax.ShapeDtypeStruct((NC, N, H), jnp.float32),
        jax.ShapeDtypeStruct((NC * RR, CC), jnp.float32),
    ],
    mesh=_mesh,
    compiler_params=_sc_params,
    scratch_types=[
        pltpu.VMEM_SHARED((N, H), jnp.float32),
        pltpu.VMEM_SHARED((RR, CC), jnp.float32),
        pltpu.VMEM((N,), jnp.float32),
        pltpu.VMEM((RR, CC), jnp.float32),
        [pltpu.VMEM((ECHUNK,), jnp.int32)] * NB,
        [pltpu.VMEM((ECHUNK,), jnp.int32)] * NB,
        [pltpu.VMEM((ECHUNK, H), jnp.float32)] * NB,
        pltpu.VMEM((TAIL_PT,), jnp.int32),
        pltpu.VMEM((TAIL_PT,), jnp.int32),
        [pltpu.SemaphoreType.DMA] * NB,
        [pltpu.SemaphoreType.DMA] * NB,
        [pltpu.SemaphoreType.DMA] * NB,
        [pltpu.SemaphoreType.DMA] * NB,
    ],
)
def _main_kernel(yhat_hbm, norm_hbm, src_hbm, dst_hbm, S_out, s_out,
                 acc_sh, s_sh, norm_v, s_v, srcb, dstb, rows, tsrc, tdst,
                 sem_si, sem_di, sem_g, sem_s):
    c = lax.axis_index("c")
    s = lax.axis_index("s")
    wid = s * NC + c
    ebase = wid * (NCH * ECHUNK)

    def idx_start(j, b):
        base = ebase + j * ECHUNK
        pltpu.async_copy(src_hbm.at[pl.ds(base, ECHUNK)], srcb[b], sem_si[b])
        pltpu.async_copy(dst_hbm.at[pl.ds(base, ECHUNK)], dstb[b], sem_di[b])

    def idx_wait(j, b):
        base = ebase + j * ECHUNK
        pltpu.make_async_copy(src_hbm.at[pl.ds(base, ECHUNK)], srcb[b],
                              sem_si[b]).wait()
        pltpu.make_async_copy(dst_hbm.at[pl.ds(base, ECHUNK)], dstb[b],
                              sem_di[b]).wait()

    def gather_start(b):
        pltpu.async_copy(yhat_hbm.at[srcb[b]], rows[b], sem_g[b])

    def gather_wait(b):
        pltpu.make_async_copy(yhat_hbm.at[srcb[b]], rows[b], sem_g[b]).wait()

    def scatter_start(b):
        pltpu.async_copy(rows[b], acc_sh.at[dstb[b]], sem_s[b], add=True)

    def scatter_wait(b):
        pltpu.make_async_copy(rows[b], acc_sh.at[dstb[b]], sem_s[b]).wait()

    def s_pass(b):
        for g in range(ECHUNK // L):
            d16 = dstb[b][pl.ds(g * L, L)]
            s16 = srcb[b][pl.ds(g * L, L)]
            val = plsc.load_gather(norm_v, [d16])
            plsc.addupdate_scatter(s_v, [s16 >> 7, s16 & 127], val)

    idx_start(0, 0)
    idx_start(1, 1)
    off = pl.multiple_of(s * STRIPE, 8)
    pltpu.sync_copy(yhat_hbm.at[pl.ds(off, STRIPE)],
                    acc_sh.at[pl.ds(off, STRIPE)])

    @pl.when(s == 0)
    def _():
        pltpu.sync_copy(yhat_hbm.at[pl.ds(TAIL_OFF, TAIL)],
                        acc_sh.at[pl.ds(TAIL_OFF, TAIL)])
    pltpu.sync_copy(norm_hbm, norm_v)
    _zero_2d(s_v)
    pltpu.sync_copy(s_v.at[pl.ds(0, RR // NS)],
                    s_sh.at[pl.ds(s * (RR // NS), RR // NS)])
    plsc.subcore_barrier()

    idx_wait(0, 0)
    gather_start(0)

    def quad_body(p, carry):
        for b in range(NB):
            j = NB * p + b
            b1 = (b + 1) % NB
            b2 = (b + 2) % NB

            @pl.when(j >= 2)
            def _():
                scatter_wait(b2)

            @pl.when(j + 2 < NCH)
            def _():
                idx_start(j + 2, b2)

            @pl.when(j + 1 < NCH)
            def _():
                idx_wait(j + 1, b1)
                gather_start(b1)

            gather_wait(b)
            scatter_start(b)
            s_pass(b)
        return carry

    lax.fori_loop(0, NCH // NB, quad_body, 0)

    scatter_wait((NCH - 2) % NB)
    scatter_wait((NCH - 1) % NB)

    tbase = E_MAIN + wid * TAIL_PT
    pltpu.sync_copy(src_hbm.at[pl.ds(tbase, TAIL_PT)], tsrc)
    pltpu.sync_copy(dst_hbm.at[pl.ds(tbase, TAIL_PT)], tdst)
    pltpu.async_copy(yhat_hbm.at[tsrc], rows[0].at[pl.ds(0, TAIL_PT)],
                     sem_g[0]).wait()
    pltpu.sync_copy(rows[0].at[pl.ds(0, TAIL_PT)], acc_sh.at[tdst], add=True)
    td16 = tdst[pl.ds(0, L)]
    ts16 = tsrc[pl.ds(0, L)]
    tval = plsc.load_gather(norm_v, [td16])
    plsc.addupdate_scatter(s_v, [ts16 >> 7, ts16 & 127], tval)

    _rowadd_to_shared(s_v, s_sh)
    plsc.subcore_barrier()
    pltpu.sync_copy(acc_sh.at[pl.ds(off, STRIPE)],
                    S_out.at[c, pl.ds(off, STRIPE)])
    _readout_shared(s_sh, s_out, c, s)

    @pl.when(s == 0)
    def _():
        pltpu.sync_copy(acc_sh.at[pl.ds(TAIL_OFF, TAIL)],
                        S_out.at[c, pl.ds(TAIL_OFF, TAIL)])


def _final_body(S_ref, yhat_ref, norm_ref, sp_ref, b1_ref, w2_ref, b2_ref,
                wp_ref, bp_ref, out_ref, acc):
    i = pl.program_id(0)

    @pl.when(i == 0)
    def _():
        acc[...] = jnp.zeros_like(acc)

    nrm = norm_ref[...]
    ssum = S_ref[0] + S_ref[1] - yhat_ref[...]
    agg = ssum * nrm + b1_ref[...]
    h = jnp.maximum(agg, 0.0)
    stot = jnp.sum(sp_ref[...], axis=1)
    cvec = nrm[:, 0] * (nrm[:, 0] + stot)
    acc[...] += jnp.dot(cvec[None, :], h, preferred_element_type=jnp.float32)

    @pl.when(i == pl.num_programs(0) - 1)
    def _():
        graph = jnp.dot(acc[...] / N, w2_ref[...],
                        preferred_element_type=jnp.float32) + b2_ref[...]
        proj = jnp.dot(graph, wp_ref[...],
                       preferred_element_type=jnp.float32) + bp_ref[...]
        nn = jnp.sqrt(jnp.sum(proj * proj))
        out_ref[...] = proj / jnp.maximum(nn, 1e-12)


_final_call = pl.pallas_call(
    _final_body,
    grid=(GRID,),
    in_specs=[
        pl.BlockSpec((NC, BN, H), lambda i: (0, i, 0)),
        pl.BlockSpec((BN, H), lambda i: (i, 0)),
        pl.BlockSpec((BN, 1), lambda i: (i, 0)),
        pl.BlockSpec((BN, NC), lambda i: (i, 0)),
        pl.BlockSpec((1, H), lambda i: (0, 0)),
        pl.BlockSpec((H, H), lambda i: (0, 0)),
        pl.BlockSpec((1, H), lambda i: (0, 0)),
        pl.BlockSpec((H, H), lambda i: (0, 0)),
        pl.BlockSpec((1, H), lambda i: (0, 0)),
    ],
    out_specs=pl.BlockSpec((1, H), lambda i: (0, 0)),
    out_shape=jax.ShapeDtypeStruct((1, H), jnp.float32),
    scratch_shapes=[pltpu.VMEM((1, H), jnp.float32)],
)


def kernel(x, edge_index, W1, b1, W2, b2, Wp, bp):
    src = edge_index[0]
    dst = edge_index[1]
    deg_lin = _deg_kernel(dst)
    degT = deg_lin.reshape(NC, NP).T
    norm_col, yhat = _prep_call(degT, x, W1)
    S_parts, s_lin = _main_kernel(yhat, norm_col.reshape(N), src, dst)
    sT = s_lin.reshape(NC, NP).T
    return _final_call(S_parts, yhat, norm_col, sT,
                       b1.reshape(1, H), W2, b2.reshape(1, H),
                       Wp, bp.reshape(1, H))
